# Initial kernel scaffold; baseline (speedup 1.0000x reference)
import jax
import jax.numpy as jnp
from jax import lax
from jax.experimental import pallas as pl
from jax.experimental.pallas import tpu as pltpu
from jax.experimental.pallas import tpu_sc as plsc

B = 4096
L = 50
E = 64
NC = 2
NS = 16
NW = NC * NS
BPW = B // NW
CH = 8
NCH = BPW // CH
NG = 4
PADROWS = CH * L + 16


def _body(center_hbm, ctxidx_hbm, ctable_hbm, xtable_hbm, out_hbm,
          cidx_v, ctxidx_v, crows_v, buf0, buf1, outbuf, sem0, sem1):
    c = lax.axis_index("c")
    s = lax.axis_index("s")
    wid = s * NC + c
    base = wid * BPW

    pltpu.sync_copy(center_hbm.at[pl.ds(base, BPW)], cidx_v)
    pltpu.sync_copy(ctxidx_hbm.at[pl.ds(base, BPW)], ctxidx_v)
    pltpu.async_copy(ctable_hbm.at[cidx_v], crows_v, sem0).wait()

    iota = lax.iota(jnp.int32, 16)

    def fire(kchunk, buf, sem):
        for jj in range(CH):
            pltpu.async_copy(xtable_hbm.at[ctxidx_v.at[kchunk * CH + jj]],
                             buf.at[pl.ds(jj * L, L)], sem)

    def drain(buf, sem):
        for jj in range(CH):
            pltpu.make_async_copy(xtable_hbm.at[ctxidx_v.at[jj]],
                                  buf.at[pl.ds(jj * L, L)], sem).wait()

    def compute(kchunk, buf):
        for jj in range(CH):
            jglob = kchunk * CH + jj
            rows = [jj * L + g * 16 + iota for g in range(NG)]

            def e_body(e, accs, rows=rows, jglob=jglob):
                cval = crows_v[jglob, e]
                col = jnp.full((16,), e, jnp.int32)
                return tuple(
                    a + cval * plsc.load_gather(buf, [rows[g], col])
                    for g, a in enumerate(accs))

            accs = lax.fori_loop(
                0, E, e_body,
                tuple(jnp.zeros((16,), jnp.float32) for _ in range(NG)))
            jvec = jnp.full((16,), jj, jnp.int32)
            for g in range(NG):
                cols = g * 16 + iota
                plsc.store_scatter(outbuf, [jvec, cols], accs[g],
                                   mask=cols < L)
        pltpu.sync_copy(outbuf, out_hbm.at[pl.ds(base + kchunk * CH, CH)])

    fire(0, buf0, sem0)

    def k2body(k2, carry):
        kc = 2 * k2
        fire(kc + 1, buf1, sem1)
        drain(buf0, sem0)
        compute(kc, buf0)

        @pl.when(kc + 2 < NCH)
        def _():
            fire(kc + 2, buf0, sem0)

        drain(buf1, sem1)
        compute(kc + 1, buf1)
        return carry

    lax.fori_loop(0, NCH // 2, k2body, 0)


_mesh = plsc.VectorSubcoreMesh(core_axis_name="c", subcore_axis_name="s")

_sc_call = pl.kernel(
    _body,
    out_type=jax.ShapeDtypeStruct((B, L), jnp.float32),
    mesh=_mesh,
    scratch_types=[
        pltpu.VMEM((BPW,), jnp.int32),
        pltpu.VMEM((BPW, L), jnp.int32),
        pltpu.VMEM((BPW, E), jnp.float32),
        pltpu.VMEM((PADROWS, E), jnp.float32),
        pltpu.VMEM((PADROWS, E), jnp.float32),
        pltpu.VMEM((CH, L), jnp.float32),
        pltpu.SemaphoreType.DMA,
        pltpu.SemaphoreType.DMA,
    ],
)


@jax.jit
def kernel(center, context_negative, center_table, context_table):
    return _sc_call(center.reshape(B), context_negative,
                    center_table, context_table)


# speedup vs baseline: 2.6513x; 2.6513x over previous
import jax
import jax.numpy as jnp
from jax import lax
from jax.experimental import pallas as pl
from jax.experimental.pallas import tpu as pltpu
from jax.experimental.pallas import tpu_sc as plsc

B = 4096
L = 50
E = 64
NC = 2
NS = 16
NW = NC * NS
BPW = B // NW
CH = 8
NCH = BPW // CH
NG = 4
PADROWS = CH * L + 16


def _body(center_hbm, ctxidx_hbm, ctable_hbm, xtable_hbm, out_hbm,
          cidx_v, ctxidx_v, crows_v, buf0, buf1, outbuf, sem0, sem1):
    c = lax.axis_index("c")
    s = lax.axis_index("s")
    wid = s * NC + c
    base = wid * BPW

    pltpu.sync_copy(center_hbm.at[pl.ds(base, BPW)], cidx_v)
    pltpu.sync_copy(ctxidx_hbm.at[pl.ds(base, BPW)], ctxidx_v)
    pltpu.async_copy(ctable_hbm.at[cidx_v], crows_v, sem0).wait()

    iota = lax.iota(jnp.int32, 16)

    def fire(kchunk, buf, sem):
        for jj in range(CH):
            pltpu.async_copy(xtable_hbm.at[ctxidx_v.at[kchunk * CH + jj]],
                             buf.at[pl.ds(jj * L, L)], sem)

    def drain(buf, sem):
        for jj in range(CH):
            pltpu.make_async_copy(xtable_hbm.at[ctxidx_v.at[jj]],
                                  buf.at[pl.ds(jj * L, L)], sem).wait()

    def compute(kchunk, buf):
        for jj in range(CH):
            jglob = kchunk * CH + jj
            rows = [jj * L + g * 16 + iota for g in range(NG)]

            def eb_body(eb, accs, rows=rows, jglob=jglob):
                cvec = crows_v[jglob, pl.ds(eb * 16, 16)]
                colbase = jnp.full((16,), eb * 16, jnp.int32)
                for ee in range(16):
                    cs = cvec[ee]
                    col = colbase + ee
                    accs = tuple(
                        a + cs * plsc.load_gather(buf, [rows[g], col])
                        for g, a in enumerate(accs))
                return accs

            accs = lax.fori_loop(
                0, E // 16, eb_body,
                tuple(jnp.zeros((16,), jnp.float32) for _ in range(NG)))
            jvec = jnp.full((16,), jj, jnp.int32)
            for g in range(NG):
                cols = g * 16 + iota
                plsc.store_scatter(outbuf, [jvec, cols], accs[g],
                                   mask=cols < L)
        pltpu.sync_copy(outbuf, out_hbm.at[pl.ds(base + kchunk * CH, CH)])

    fire(0, buf0, sem0)

    def k2body(k2, carry):
        kc = 2 * k2
        fire(kc + 1, buf1, sem1)
        drain(buf0, sem0)
        compute(kc, buf0)

        @pl.when(kc + 2 < NCH)
        def _():
            fire(kc + 2, buf0, sem0)

        drain(buf1, sem1)
        compute(kc + 1, buf1)
        return carry

    lax.fori_loop(0, NCH // 2, k2body, 0)


_mesh = plsc.VectorSubcoreMesh(core_axis_name="c", subcore_axis_name="s")

_sc_call = pl.kernel(
    _body,
    out_type=jax.ShapeDtypeStruct((B, L), jnp.float32),
    mesh=_mesh,
    scratch_types=[
        pltpu.VMEM((BPW,), jnp.int32),
        pltpu.VMEM((BPW, L), jnp.int32),
        pltpu.VMEM((BPW, E), jnp.float32),
        pltpu.VMEM((PADROWS, E), jnp.float32),
        pltpu.VMEM((PADROWS, E), jnp.float32),
        pltpu.VMEM((CH, L), jnp.float32),
        pltpu.SemaphoreType.DMA,
        pltpu.SemaphoreType.DMA,
    ],
    compiler_params=pltpu.CompilerParams(needs_layout_passes=False, use_tc_tiling_on_sc=False),
)


@jax.jit
def kernel(center, context_negative, center_table, context_table):
    return _sc_call(center.reshape(B), context_negative,
                    center_table, context_table)


# speedup vs baseline: 6.6568x; 2.5108x over previous
import jax
import jax.numpy as jnp
from jax import lax
from jax.experimental import pallas as pl
from jax.experimental.pallas import tpu as pltpu
from jax.experimental.pallas import tpu_sc as plsc

B = 4096
L = 50
E = 64
NC = 2
NS = 16
NW = NC * NS
BPW = B // NW
CH = 8
NCH = BPW // CH
NG = 4
PADROWS = CH * L + 16


def _body(center_hbm, ctxidx_hbm, ctable_hbm, xtable_hbm, out_hbm,
          cidx_v, ctxidx_v, crows_v, buf0, buf1, outbuf, sem0, sem1):
    c = lax.axis_index("c")
    s = lax.axis_index("s")
    wid = s * NC + c
    base = wid * BPW

    pltpu.sync_copy(center_hbm.at[pl.ds(base, BPW)], cidx_v)
    pltpu.sync_copy(ctxidx_hbm.at[pl.ds(base, BPW)], ctxidx_v)
    pltpu.async_copy(ctable_hbm.at[cidx_v], crows_v, sem0).wait()

    iota = lax.iota(jnp.int32, 16)

    def fire(kchunk, buf, sem):
        for jj in range(CH):
            pltpu.async_copy(xtable_hbm.at[ctxidx_v.at[kchunk * CH + jj]],
                             buf.at[pl.ds(jj * L, L)], sem)

    def drain(buf, sem):
        for jj in range(CH):
            pltpu.make_async_copy(xtable_hbm.at[ctxidx_v.at[jj]],
                                  buf.at[pl.ds(jj * L, L)], sem).wait()

    def compute(kchunk, buf):
        def jj_body(jj, carry):
            jglob = kchunk * CH + jj
            row0 = jj * L
            cv = [crows_v[jglob, pl.ds(i * 16, 16)] for i in range(E // 16)]
            outs = [jnp.zeros((16,), jnp.float32) for _ in range(NG)]
            for l in range(L):
                row = row0 + l
                p = cv[0] * buf[row, pl.ds(0, 16)]
                for i in range(1, E // 16):
                    p = p + cv[i] * buf[row, pl.ds(i * 16, 16)]
                s = jnp.sum(p)
                g, ll = divmod(l, 16)
                outs[g] = jnp.where(iota == ll, s, outs[g])
            jvec = jnp.full((16,), jj, jnp.int32)
            for g in range(NG):
                cols = g * 16 + iota
                plsc.store_scatter(outbuf, [jvec, cols], outs[g],
                                   mask=cols < L)
            return carry
        lax.fori_loop(0, CH, jj_body, 0)
        pltpu.sync_copy(outbuf, out_hbm.at[pl.ds(base + kchunk * CH, CH)])

    fire(0, buf0, sem0)

    def k2body(k2, carry):
        kc = 2 * k2
        fire(kc + 1, buf1, sem1)
        drain(buf0, sem0)
        compute(kc, buf0)

        @pl.when(kc + 2 < NCH)
        def _():
            fire(kc + 2, buf0, sem0)

        drain(buf1, sem1)
        compute(kc + 1, buf1)
        return carry

    lax.fori_loop(0, NCH // 2, k2body, 0)


_mesh = plsc.VectorSubcoreMesh(core_axis_name="c", subcore_axis_name="s")

_sc_call = pl.kernel(
    _body,
    out_type=jax.ShapeDtypeStruct((B, L), jnp.float32),
    mesh=_mesh,
    scratch_types=[
        pltpu.VMEM((BPW,), jnp.int32),
        pltpu.VMEM((BPW, L), jnp.int32),
        pltpu.VMEM((BPW, E), jnp.float32),
        pltpu.VMEM((PADROWS, E), jnp.float32),
        pltpu.VMEM((PADROWS, E), jnp.float32),
        pltpu.VMEM((CH, L), jnp.float32),
        pltpu.SemaphoreType.DMA,
        pltpu.SemaphoreType.DMA,
    ],
    compiler_params=pltpu.CompilerParams(needs_layout_passes=False, use_tc_tiling_on_sc=False),
)


@jax.jit
def kernel(center, context_negative, center_table, context_table):
    return _sc_call(center.reshape(B), context_negative,
                    center_table, context_table)
